# baseline (speedup 1.0000x reference)
import math

import jax
import jax.numpy as jnp
import numpy as np
from jax.experimental import pallas as pl

S = 2048
D = 1024
H = 16
P = 8
HD = D // H
DFD = D // 4
T = 256
SB = S // T
TA = 64
WA = 256
NA = S // TA
HALO = 144
MAX_DIST = 128.0
OFFSET_SCALE = 8.0
SCALE = HD ** -0.5
BOUNDS = (0.0, 0.5, 1.0, 2.0, 4.0, 8.0, 12.0, 24.0)


def _proj_kernel(x_ref, w_ref, b_ref, o_ref):
    o_ref[:, :] = (
        jnp.dot(x_ref[:, :], w_ref[:, :], preferred_element_type=jnp.float32)
        + b_ref[0]
    )


def _tc_kernel(tcr_ref, dt_ref, wdc_ref, ediff_ref, woff_ref, wps_ref,
               bop_ref, off_ref, ps_ref):
    dt = jnp.maximum(dt_ref[:, :], 0.0)
    dl = jnp.log1p(dt)
    tc = tcr_ref[:, :] + dl * wdc_ref[0]
    for j in range(8):
        mask = (dt > BOUNDS[j]).astype(jnp.float32)
        tc = tc + mask * ediff_ref[j]
    off_ref[:, :] = jnp.tanh(
        jnp.dot(tc, woff_ref[:, :], preferred_element_type=jnp.float32)
        + bop_ref[0]
    )
    ps_ref[:, :] = (
        jnp.dot(tc, wps_ref[:, :], preferred_element_type=jnp.float32)
        + bop_ref[1]
    )


def _band_reduce(ms, vec):
    cols = []
    for p in range(P):
        s = ms[p] * vec
        h = s[:, :128] + s[:, 128:256]
        for k in range(2, WA // 128):
            h = h + s[:, 128 * k:128 * (k + 1)]
        cols.append(jnp.sum(h, axis=1, keepdims=True))
    return jnp.concatenate(cols, axis=1)


def _attn_kernel(q_ref, k_ref, v_ref, off_ref, ps_ref, elw_ref, elq_ref,
                 anch_ref, dc_ref, cb_ref, o_ref):
    sb = pl.program_id(1)
    t0 = sb * TA
    start = jnp.clip(t0 - HALO, 0, S - WA)

    q2 = q_ref[:, :] * SCALE
    kw2 = k_ref[pl.ds(start, WA), :]
    vw2 = v_ref[pl.ds(start, WA), :]

    qpos = (t0 + jax.lax.broadcasted_iota(jnp.int32, (TA, 1), 0)).astype(
        jnp.float32)
    jfw = (start + jax.lax.broadcasted_iota(jnp.int32, (1, WA), 1)
           ).astype(jnp.float32)
    lane = jax.lax.broadcasted_iota(jnp.int32, (1, 128), 1)
    elv2 = elw_ref[0]
    elq = elq_ref[:, :]

    out = None
    for hh in range(2):
        mask = ((lane >= 64 * hh) & (lane < 64 * (hh + 1))).astype(jnp.float32)
        off = off_ref[hh]
        ps = ps_ref[hh]

        sp = qpos - anch_ref[:, :] + off * OFFSET_SCALE
        sp = jnp.maximum(sp, 0.0)
        sp = jnp.minimum(sp, qpos)

        m_list = [
            jnp.maximum(1.0 - jnp.abs(jfw - sp[:, p:p + 1]), 0.0)
            for p in range(P)
        ]

        sd = jax.lax.dot_general(q2, kw2 * mask, (((1,), (1,)), ((), ())),
                                 preferred_element_type=jnp.float32)
        qk = _band_reduce(m_list, sd)
        sel = _band_reduce(m_list, elv2)
        rel = jnp.log1p(jnp.maximum(elq - sel, 0.0))

        logits = qk + ps + cb_ref[hh] - dc_ref[hh] * rel
        mx = jnp.max(logits, axis=1, keepdims=True)
        ex = jnp.exp(logits - mx)
        wgt = ex / jnp.sum(ex, axis=1, keepdims=True)

        a = wgt[:, 0:1] * m_list[0]
        for p in range(1, P):
            a = a + wgt[:, p:p + 1] * m_list[p]
        oh = jnp.dot(a, vw2 * mask, preferred_element_type=jnp.float32)
        out = oh if out is None else out + oh
    o_ref[:, :] = out


def _out_kernel(a_ref, w_ref, b_ref, o_ref):
    o_ref[:, :] = (
        jnp.dot(a_ref[:, :], w_ref[:, :], preferred_element_type=jnp.float32)
        + b_ref[:, :]
    )


def kernel(x, time_delta, Wq, bq, Wk, bk, Wv, bv, emb, Wdc, bdc, Wtc, btc,
           Woff, boff, Wps, bps, Wo, bo, point_bias, tdw, tdb):
    f32 = jnp.float32
    x2 = x[0]
    dt_raw = time_delta[0].reshape(S, 1)

    Wtc_x = Wtc[:D]
    Wtc_c = Wtc[D:D + DFD]
    Wtc_b = Wtc[D + DFD:]
    wdc_eff = (Wdc @ Wtc_c).reshape(1, 1, D)
    emb_eff = emb @ Wtc_b
    tc_bias = btc + bdc @ Wtc_c + emb_eff[0]
    emb_diff = (emb_eff[1:] - emb_eff[:-1]).reshape(8, 1, D)

    Wbig = jnp.concatenate([Wq, Wk, Wv, Wtc_x], axis=1)
    bias4 = jnp.stack([bq, bk, bv, tc_bias]).reshape(4, 1, D)

    qkvt = pl.pallas_call(
        _proj_kernel,
        grid=(4, SB),
        in_specs=[
            pl.BlockSpec((T, D), lambda nb, sb: (sb, 0)),
            pl.BlockSpec((D, D), lambda nb, sb: (0, nb)),
            pl.BlockSpec((1, 1, D), lambda nb, sb: (nb, 0, 0)),
        ],
        out_specs=pl.BlockSpec((T, D), lambda nb, sb: (sb, nb)),
        out_shape=jax.ShapeDtypeStruct((S, 4 * D), f32),
    )(x2, Wbig, bias4)

    bop = jnp.stack([boff, bps]).reshape(2, 1, H * P)
    off, pscore = pl.pallas_call(
        _tc_kernel,
        grid=(SB,),
        in_specs=[
            pl.BlockSpec((T, D), lambda sb: (sb, 3)),
            pl.BlockSpec((T, 1), lambda sb: (sb, 0)),
            pl.BlockSpec((1, 1, D), lambda sb: (0, 0, 0)),
            pl.BlockSpec((8, 1, D), lambda sb: (0, 0, 0)),
            pl.BlockSpec((D, H * P), lambda sb: (0, 0)),
            pl.BlockSpec((D, H * P), lambda sb: (0, 0)),
            pl.BlockSpec((2, 1, H * P), lambda sb: (0, 0, 0)),
        ],
        out_specs=[
            pl.BlockSpec((T, H * P), lambda sb: (sb, 0)),
            pl.BlockSpec((T, H * P), lambda sb: (sb, 0)),
        ],
        out_shape=[
            jax.ShapeDtypeStruct((S, H * P), f32),
            jax.ShapeDtypeStruct((S, H * P), f32),
        ],
    )(qkvt, dt_raw, wdc_eff, emb_diff, Woff, Wps, bop)

    off_t = off.reshape(S, H, P).transpose(1, 0, 2)
    ps_t = pscore.reshape(S, H, P).transpose(1, 0, 2)

    dtc = jnp.maximum(time_delta[0], 0.0)
    elapsed = jnp.cumsum(dtc)
    elq = elapsed.reshape(S, 1)
    starts = [min(max(0, TA * sb - HALO), S - WA) for sb in range(NA)]
    elw = jnp.stack(
        [elapsed[s0:s0 + WA] for s0 in starts]).reshape(NA, 1, WA)

    anchors = 2.0 ** jnp.linspace(0.0, math.log2(MAX_DIST + 1.0), P) - 1.0
    anchors = anchors.at[0].set(0.0).astype(f32)
    anch1 = anchors.reshape(1, P)
    dc = jax.nn.softplus(tdw).reshape(H, 1, P)
    cb = (tdb + point_bias).reshape(H, 1, P)

    attn = pl.pallas_call(
        _attn_kernel,
        grid=(H // 2, NA),
        in_specs=[
            pl.BlockSpec((TA, 128), lambda hp, sb: (sb, hp)),
            pl.BlockSpec((S, 128), lambda hp, sb: (0, 8 + hp)),
            pl.BlockSpec((S, 128), lambda hp, sb: (0, 16 + hp)),
            pl.BlockSpec((2, TA, P), lambda hp, sb: (hp, sb, 0)),
            pl.BlockSpec((2, TA, P), lambda hp, sb: (hp, sb, 0)),
            pl.BlockSpec((1, 1, WA), lambda hp, sb: (sb, 0, 0)),
            pl.BlockSpec((TA, 1), lambda hp, sb: (sb, 0)),
            pl.BlockSpec((1, P), lambda hp, sb: (0, 0)),
            pl.BlockSpec((2, 1, P), lambda hp, sb: (hp, 0, 0)),
            pl.BlockSpec((2, 1, P), lambda hp, sb: (hp, 0, 0)),
        ],
        out_specs=pl.BlockSpec((TA, 128), lambda hp, sb: (sb, hp)),
        out_shape=jax.ShapeDtypeStruct((S, D), f32),
    )(qkvt, qkvt, qkvt, off_t, ps_t, elw, elq, anch1, dc, cb)

    out = pl.pallas_call(
        _out_kernel,
        grid=(SB,),
        in_specs=[
            pl.BlockSpec((T, D), lambda sb: (sb, 0)),
            pl.BlockSpec((D, D), lambda sb: (0, 0)),
            pl.BlockSpec((1, D), lambda sb: (0, 0)),
        ],
        out_specs=pl.BlockSpec((T, D), lambda sb: (sb, 0)),
        out_shape=jax.ShapeDtypeStruct((S, D), f32),
    )(attn, Wo, bo.reshape(1, D))

    return out.reshape(1, S, D)


# speedup vs baseline: 1.1766x; 1.1766x over previous
import math

import jax
import jax.numpy as jnp
import numpy as np
from jax.experimental import pallas as pl

S = 2048
D = 1024
H = 16
P = 8
HD = D // H
DFD = D // 4
T = 256
SB = S // T
TA = 128
WA = 384
NA = S // TA
HALO = 144
MAX_DIST = 128.0
OFFSET_SCALE = 8.0
SCALE = HD ** -0.5
BOUNDS = (0.0, 0.5, 1.0, 2.0, 4.0, 8.0, 12.0, 24.0)


def _proj_kernel(x_ref, w_ref, b_ref, o_ref):
    o_ref[:, :] = (
        jnp.dot(x_ref[:, :], w_ref[:, :], preferred_element_type=jnp.float32)
        + b_ref[0]
    )


def _tc_kernel(tcr_ref, dt_ref, wdc_ref, ediff_ref, woff_ref, wps_ref,
               bop_ref, off_ref, ps_ref):
    dt = jnp.maximum(dt_ref[:, :], 0.0)
    dl = jnp.log1p(dt)
    tc = tcr_ref[:, :] + dl * wdc_ref[0]
    for j in range(8):
        mask = (dt > BOUNDS[j]).astype(jnp.float32)
        tc = tc + mask * ediff_ref[j]
    off_ref[:, :] = jnp.tanh(
        jnp.dot(tc, woff_ref[:, :], preferred_element_type=jnp.float32)
        + bop_ref[0]
    )
    ps_ref[:, :] = (
        jnp.dot(tc, wps_ref[:, :], preferred_element_type=jnp.float32)
        + bop_ref[1]
    )


def _band_reduce(ms, vec):
    cols = []
    for p in range(P):
        s = ms[p] * vec
        h = s[:, :128] + s[:, 128:256]
        for k in range(2, WA // 128):
            h = h + s[:, 128 * k:128 * (k + 1)]
        cols.append(jnp.sum(h, axis=1, keepdims=True))
    return jnp.concatenate(cols, axis=1)


def _attn_kernel(q_ref, k_ref, v_ref, off_ref, ps_ref, elw_ref, elq_ref,
                 anch_ref, dc_ref, cb_ref, o_ref):
    sb = pl.program_id(1)
    t0 = sb * TA
    start = jnp.clip(t0 - HALO, 0, S - WA)

    q2 = q_ref[:, :] * SCALE
    kw2 = k_ref[pl.ds(start, WA), :]
    vw2 = v_ref[pl.ds(start, WA), :]

    qpos = (t0 + jax.lax.broadcasted_iota(jnp.int32, (TA, 1), 0)).astype(
        jnp.float32)
    jfw = (start + jax.lax.broadcasted_iota(jnp.int32, (1, WA), 1)
           ).astype(jnp.float32)
    lane = jax.lax.broadcasted_iota(jnp.int32, (1, 128), 1)
    elv2 = elw_ref[0]
    elq = elq_ref[:, :]

    out = None
    for hh in range(2):
        mask = ((lane >= 64 * hh) & (lane < 64 * (hh + 1))).astype(jnp.float32)
        off = off_ref[hh]
        ps = ps_ref[hh]

        sp = qpos - anch_ref[:, :] + off * OFFSET_SCALE
        sp = jnp.maximum(sp, 0.0)
        sp = jnp.minimum(sp, qpos)

        m_list = [
            jnp.maximum(1.0 - jnp.abs(jfw - sp[:, p:p + 1]), 0.0)
            for p in range(P)
        ]

        sd = jax.lax.dot_general(q2, kw2 * mask, (((1,), (1,)), ((), ())),
                                 preferred_element_type=jnp.float32)
        qk = _band_reduce(m_list, sd)
        sel = _band_reduce(m_list, elv2)
        rel = jnp.log1p(jnp.maximum(elq - sel, 0.0))

        logits = qk + ps + cb_ref[hh] - dc_ref[hh] * rel
        mx = jnp.max(logits, axis=1, keepdims=True)
        ex = jnp.exp(logits - mx)
        wgt = ex / jnp.sum(ex, axis=1, keepdims=True)

        a = wgt[:, 0:1] * m_list[0]
        for p in range(1, P):
            a = a + wgt[:, p:p + 1] * m_list[p]
        oh = jnp.dot(a, vw2 * mask, preferred_element_type=jnp.float32)
        out = oh if out is None else out + oh
    o_ref[:, :] = out


def _out_kernel(a_ref, w_ref, b_ref, o_ref):
    o_ref[:, :] = (
        jnp.dot(a_ref[:, :], w_ref[:, :], preferred_element_type=jnp.float32)
        + b_ref[:, :]
    )


def kernel(x, time_delta, Wq, bq, Wk, bk, Wv, bv, emb, Wdc, bdc, Wtc, btc,
           Woff, boff, Wps, bps, Wo, bo, point_bias, tdw, tdb):
    f32 = jnp.float32
    x2 = x[0]
    dt_raw = time_delta[0].reshape(S, 1)

    Wtc_x = Wtc[:D]
    Wtc_c = Wtc[D:D + DFD]
    Wtc_b = Wtc[D + DFD:]
    wdc_eff = (Wdc @ Wtc_c).reshape(1, 1, D)
    emb_eff = emb @ Wtc_b
    tc_bias = btc + bdc @ Wtc_c + emb_eff[0]
    emb_diff = (emb_eff[1:] - emb_eff[:-1]).reshape(8, 1, D)

    Wbig = jnp.concatenate([Wq, Wk, Wv, Wtc_x], axis=1)
    bias4 = jnp.stack([bq, bk, bv, tc_bias]).reshape(4, 1, D)

    qkvt = pl.pallas_call(
        _proj_kernel,
        grid=(4, SB),
        in_specs=[
            pl.BlockSpec((T, D), lambda nb, sb: (sb, 0)),
            pl.BlockSpec((D, D), lambda nb, sb: (0, nb)),
            pl.BlockSpec((1, 1, D), lambda nb, sb: (nb, 0, 0)),
        ],
        out_specs=pl.BlockSpec((T, D), lambda nb, sb: (sb, nb)),
        out_shape=jax.ShapeDtypeStruct((S, 4 * D), f32),
    )(x2, Wbig, bias4)

    bop = jnp.stack([boff, bps]).reshape(2, 1, H * P)
    off, pscore = pl.pallas_call(
        _tc_kernel,
        grid=(SB,),
        in_specs=[
            pl.BlockSpec((T, D), lambda sb: (sb, 3)),
            pl.BlockSpec((T, 1), lambda sb: (sb, 0)),
            pl.BlockSpec((1, 1, D), lambda sb: (0, 0, 0)),
            pl.BlockSpec((8, 1, D), lambda sb: (0, 0, 0)),
            pl.BlockSpec((D, H * P), lambda sb: (0, 0)),
            pl.BlockSpec((D, H * P), lambda sb: (0, 0)),
            pl.BlockSpec((2, 1, H * P), lambda sb: (0, 0, 0)),
        ],
        out_specs=[
            pl.BlockSpec((T, H * P), lambda sb: (sb, 0)),
            pl.BlockSpec((T, H * P), lambda sb: (sb, 0)),
        ],
        out_shape=[
            jax.ShapeDtypeStruct((S, H * P), f32),
            jax.ShapeDtypeStruct((S, H * P), f32),
        ],
    )(qkvt, dt_raw, wdc_eff, emb_diff, Woff, Wps, bop)

    off_t = off.reshape(S, H, P).transpose(1, 0, 2)
    ps_t = pscore.reshape(S, H, P).transpose(1, 0, 2)

    dtc = jnp.maximum(time_delta[0], 0.0)
    elapsed = jnp.cumsum(dtc)
    elq = elapsed.reshape(S, 1)
    starts = [min(max(0, TA * sb - HALO), S - WA) for sb in range(NA)]
    elw = jnp.stack(
        [elapsed[s0:s0 + WA] for s0 in starts]).reshape(NA, 1, WA)

    anchors = 2.0 ** jnp.linspace(0.0, math.log2(MAX_DIST + 1.0), P) - 1.0
    anchors = anchors.at[0].set(0.0).astype(f32)
    anch1 = anchors.reshape(1, P)
    dc = jax.nn.softplus(tdw).reshape(H, 1, P)
    cb = (tdb + point_bias).reshape(H, 1, P)

    attn = pl.pallas_call(
        _attn_kernel,
        grid=(H // 2, NA),
        in_specs=[
            pl.BlockSpec((TA, 128), lambda hp, sb: (sb, hp)),
            pl.BlockSpec((S, 128), lambda hp, sb: (0, 8 + hp)),
            pl.BlockSpec((S, 128), lambda hp, sb: (0, 16 + hp)),
            pl.BlockSpec((2, TA, P), lambda hp, sb: (hp, sb, 0)),
            pl.BlockSpec((2, TA, P), lambda hp, sb: (hp, sb, 0)),
            pl.BlockSpec((1, 1, WA), lambda hp, sb: (sb, 0, 0)),
            pl.BlockSpec((TA, 1), lambda hp, sb: (sb, 0)),
            pl.BlockSpec((1, P), lambda hp, sb: (0, 0)),
            pl.BlockSpec((2, 1, P), lambda hp, sb: (hp, 0, 0)),
            pl.BlockSpec((2, 1, P), lambda hp, sb: (hp, 0, 0)),
        ],
        out_specs=pl.BlockSpec((TA, 128), lambda hp, sb: (sb, hp)),
        out_shape=jax.ShapeDtypeStruct((S, D), f32),
    )(qkvt, qkvt, qkvt, off_t, ps_t, elw, elq, anch1, dc, cb)

    out = pl.pallas_call(
        _out_kernel,
        grid=(SB,),
        in_specs=[
            pl.BlockSpec((T, D), lambda sb: (sb, 0)),
            pl.BlockSpec((D, D), lambda sb: (0, 0)),
            pl.BlockSpec((1, D), lambda sb: (0, 0)),
        ],
        out_specs=pl.BlockSpec((T, D), lambda sb: (sb, 0)),
        out_shape=jax.ShapeDtypeStruct((S, D), f32),
    )(attn, Wo, bo.reshape(1, D))

    return out.reshape(1, S, D)


# speedup vs baseline: 1.2433x; 1.0567x over previous
import math

import jax
import jax.numpy as jnp
import numpy as np
from jax.experimental import pallas as pl

S = 2048
D = 1024
H = 16
P = 8
HD = D // H
DFD = D // 4
T = 256
SB = S // T
TA = 256
WA = 512
NA = S // TA
HALO = 144
MAX_DIST = 128.0
OFFSET_SCALE = 8.0
SCALE = HD ** -0.5
BOUNDS = (0.0, 0.5, 1.0, 2.0, 4.0, 8.0, 12.0, 24.0)


def _proj_kernel(x_ref, w_ref, b_ref, o_ref):
    o_ref[:, :] = (
        jnp.dot(x_ref[:, :], w_ref[:, :], preferred_element_type=jnp.float32)
        + b_ref[0]
    )


def _tc_kernel(tcr_ref, dt_ref, wdc_ref, ediff_ref, woff_ref, wps_ref,
               bop_ref, off_ref, ps_ref):
    dt = jnp.maximum(dt_ref[:, :], 0.0)
    dl = jnp.log1p(dt)
    tc = tcr_ref[:, :] + dl * wdc_ref[0]
    for j in range(8):
        mask = (dt > BOUNDS[j]).astype(jnp.float32)
        tc = tc + mask * ediff_ref[j]
    off_ref[:, :] = jnp.tanh(
        jnp.dot(tc, woff_ref[:, :], preferred_element_type=jnp.float32)
        + bop_ref[0]
    )
    ps_ref[:, :] = (
        jnp.dot(tc, wps_ref[:, :], preferred_element_type=jnp.float32)
        + bop_ref[1]
    )


def _band_reduce(ms, vec):
    cols = []
    for p in range(P):
        s = ms[p] * vec
        h = s[:, :128] + s[:, 128:256]
        for k in range(2, WA // 128):
            h = h + s[:, 128 * k:128 * (k + 1)]
        cols.append(jnp.sum(h, axis=1, keepdims=True))
    return jnp.concatenate(cols, axis=1)


def _attn_kernel(q_ref, k_ref, v_ref, off_ref, ps_ref, elw_ref, elq_ref,
                 anch_ref, dc_ref, cb_ref, o_ref):
    sb = pl.program_id(1)
    t0 = sb * TA
    start = jnp.clip(t0 - HALO, 0, S - WA)

    q2 = q_ref[:, :] * SCALE
    kw2 = k_ref[pl.ds(start, WA), :]
    vw2 = v_ref[pl.ds(start, WA), :]

    qpos = (t0 + jax.lax.broadcasted_iota(jnp.int32, (TA, 1), 0)).astype(
        jnp.float32)
    jfw = (start + jax.lax.broadcasted_iota(jnp.int32, (1, WA), 1)
           ).astype(jnp.float32)
    lane = jax.lax.broadcasted_iota(jnp.int32, (1, 128), 1)
    elv2 = elw_ref[0]
    elq = elq_ref[:, :]

    out = None
    for hh in range(2):
        mask = ((lane >= 64 * hh) & (lane < 64 * (hh + 1))).astype(jnp.float32)
        off = off_ref[hh]
        ps = ps_ref[hh]

        sp = qpos - anch_ref[:, :] + off * OFFSET_SCALE
        sp = jnp.maximum(sp, 0.0)
        sp = jnp.minimum(sp, qpos)

        m_list = [
            jnp.maximum(1.0 - jnp.abs(jfw - sp[:, p:p + 1]), 0.0)
            for p in range(P)
        ]

        sd = jax.lax.dot_general(q2, kw2 * mask, (((1,), (1,)), ((), ())),
                                 preferred_element_type=jnp.float32)
        qk = _band_reduce(m_list, sd)
        sel = _band_reduce(m_list, elv2)
        rel = jnp.log1p(jnp.maximum(elq - sel, 0.0))

        logits = qk + ps + cb_ref[hh] - dc_ref[hh] * rel
        mx = jnp.max(logits, axis=1, keepdims=True)
        ex = jnp.exp(logits - mx)
        wgt = ex / jnp.sum(ex, axis=1, keepdims=True)

        a = wgt[:, 0:1] * m_list[0]
        for p in range(1, P):
            a = a + wgt[:, p:p + 1] * m_list[p]
        oh = jnp.dot(a, vw2 * mask, preferred_element_type=jnp.float32)
        out = oh if out is None else out + oh
    o_ref[:, :] = out


def _out_kernel(a_ref, w_ref, b_ref, o_ref):
    o_ref[:, :] = (
        jnp.dot(a_ref[:, :], w_ref[:, :], preferred_element_type=jnp.float32)
        + b_ref[:, :]
    )


def kernel(x, time_delta, Wq, bq, Wk, bk, Wv, bv, emb, Wdc, bdc, Wtc, btc,
           Woff, boff, Wps, bps, Wo, bo, point_bias, tdw, tdb):
    f32 = jnp.float32
    x2 = x[0]
    dt_raw = time_delta[0].reshape(S, 1)

    Wtc_x = Wtc[:D]
    Wtc_c = Wtc[D:D + DFD]
    Wtc_b = Wtc[D + DFD:]
    wdc_eff = (Wdc @ Wtc_c).reshape(1, 1, D)
    emb_eff = emb @ Wtc_b
    tc_bias = btc + bdc @ Wtc_c + emb_eff[0]
    emb_diff = (emb_eff[1:] - emb_eff[:-1]).reshape(8, 1, D)

    Wbig = jnp.concatenate([Wq, Wk, Wv, Wtc_x], axis=1)
    bias4 = jnp.stack([bq, bk, bv, tc_bias]).reshape(4, 1, D)

    qkvt = pl.pallas_call(
        _proj_kernel,
        grid=(4, SB),
        in_specs=[
            pl.BlockSpec((T, D), lambda nb, sb: (sb, 0)),
            pl.BlockSpec((D, D), lambda nb, sb: (0, nb)),
            pl.BlockSpec((1, 1, D), lambda nb, sb: (nb, 0, 0)),
        ],
        out_specs=pl.BlockSpec((T, D), lambda nb, sb: (sb, nb)),
        out_shape=jax.ShapeDtypeStruct((S, 4 * D), f32),
    )(x2, Wbig, bias4)

    bop = jnp.stack([boff, bps]).reshape(2, 1, H * P)
    off, pscore = pl.pallas_call(
        _tc_kernel,
        grid=(SB,),
        in_specs=[
            pl.BlockSpec((T, D), lambda sb: (sb, 3)),
            pl.BlockSpec((T, 1), lambda sb: (sb, 0)),
            pl.BlockSpec((1, 1, D), lambda sb: (0, 0, 0)),
            pl.BlockSpec((8, 1, D), lambda sb: (0, 0, 0)),
            pl.BlockSpec((D, H * P), lambda sb: (0, 0)),
            pl.BlockSpec((D, H * P), lambda sb: (0, 0)),
            pl.BlockSpec((2, 1, H * P), lambda sb: (0, 0, 0)),
        ],
        out_specs=[
            pl.BlockSpec((T, H * P), lambda sb: (sb, 0)),
            pl.BlockSpec((T, H * P), lambda sb: (sb, 0)),
        ],
        out_shape=[
            jax.ShapeDtypeStruct((S, H * P), f32),
            jax.ShapeDtypeStruct((S, H * P), f32),
        ],
    )(qkvt, dt_raw, wdc_eff, emb_diff, Woff, Wps, bop)

    off_t = off.reshape(S, H, P).transpose(1, 0, 2)
    ps_t = pscore.reshape(S, H, P).transpose(1, 0, 2)

    dtc = jnp.maximum(time_delta[0], 0.0)
    elapsed = jnp.cumsum(dtc)
    elq = elapsed.reshape(S, 1)
    starts = [min(max(0, TA * sb - HALO), S - WA) for sb in range(NA)]
    elw = jnp.stack(
        [elapsed[s0:s0 + WA] for s0 in starts]).reshape(NA, 1, WA)

    anchors = 2.0 ** jnp.linspace(0.0, math.log2(MAX_DIST + 1.0), P) - 1.0
    anchors = anchors.at[0].set(0.0).astype(f32)
    anch1 = anchors.reshape(1, P)
    dc = jax.nn.softplus(tdw).reshape(H, 1, P)
    cb = (tdb + point_bias).reshape(H, 1, P)

    attn = pl.pallas_call(
        _attn_kernel,
        grid=(H // 2, NA),
        in_specs=[
            pl.BlockSpec((TA, 128), lambda hp, sb: (sb, hp)),
            pl.BlockSpec((S, 128), lambda hp, sb: (0, 8 + hp)),
            pl.BlockSpec((S, 128), lambda hp, sb: (0, 16 + hp)),
            pl.BlockSpec((2, TA, P), lambda hp, sb: (hp, sb, 0)),
            pl.BlockSpec((2, TA, P), lambda hp, sb: (hp, sb, 0)),
            pl.BlockSpec((1, 1, WA), lambda hp, sb: (sb, 0, 0)),
            pl.BlockSpec((TA, 1), lambda hp, sb: (sb, 0)),
            pl.BlockSpec((1, P), lambda hp, sb: (0, 0)),
            pl.BlockSpec((2, 1, P), lambda hp, sb: (hp, 0, 0)),
            pl.BlockSpec((2, 1, P), lambda hp, sb: (hp, 0, 0)),
        ],
        out_specs=pl.BlockSpec((TA, 128), lambda hp, sb: (sb, hp)),
        out_shape=jax.ShapeDtypeStruct((S, D), f32),
    )(qkvt, qkvt, qkvt, off_t, ps_t, elw, elq, anch1, dc, cb)

    out = pl.pallas_call(
        _out_kernel,
        grid=(SB,),
        in_specs=[
            pl.BlockSpec((T, D), lambda sb: (sb, 0)),
            pl.BlockSpec((D, D), lambda sb: (0, 0)),
            pl.BlockSpec((1, D), lambda sb: (0, 0)),
        ],
        out_specs=pl.BlockSpec((T, D), lambda sb: (sb, 0)),
        out_shape=jax.ShapeDtypeStruct((S, D), f32),
    )(attn, Wo, bo.reshape(1, D))

    return out.reshape(1, S, D)


# speedup vs baseline: 1.2434x; 1.0000x over previous
import math

import jax
import jax.numpy as jnp
from jax.experimental import pallas as pl

S = 2048
D = 1024
H = 16
P = 8
HD = D // H
DFD = D // 4
T = 256
SB = S // T
TA = 256
WA = 512
NA = S // TA
HALO = 144
MAX_DIST = 128.0
OFFSET_SCALE = 8.0
SCALE = HD ** -0.5
BOUNDS = (0.0, 0.5, 1.0, 2.0, 4.0, 8.0, 12.0, 24.0)


def _proj_kernel(x_ref, w_ref, b_ref, o_ref):
    o_ref[:, :] = (
        jnp.dot(x_ref[:, :], w_ref[:, :], preferred_element_type=jnp.float32)
        + b_ref[0]
    )


def _tc_kernel(tcr_ref, dt_ref, wdc_ref, ediff_ref, woff_ref, wps_ref,
               bop_ref, off_ref, ps_ref):
    dt = jnp.maximum(dt_ref[:, :], 0.0)
    dl = jnp.log1p(dt)
    tc = tcr_ref[:, :] + dl * wdc_ref[0]
    for j in range(8):
        mask = (dt > BOUNDS[j]).astype(jnp.float32)
        tc = tc + mask * ediff_ref[j]
    off_ref[:, :] = jnp.tanh(
        jnp.dot(tc, woff_ref[:, :], preferred_element_type=jnp.float32)
        + bop_ref[0]
    )
    ps_ref[:, :] = (
        jnp.dot(tc, wps_ref[:, :], preferred_element_type=jnp.float32)
        + bop_ref[1]
    )


def _band_reduce(ms, vec):
    cols = []
    for p in range(P):
        s = ms[p] * vec
        h = s[:, :128] + s[:, 128:256]
        for k in range(2, WA // 128):
            h = h + s[:, 128 * k:128 * (k + 1)]
        cols.append(jnp.sum(h, axis=1, keepdims=True))
    return jnp.concatenate(cols, axis=1)


def _attn_kernel(q_ref, k_ref, v_ref, off_ref, ps_ref, elw_ref, elq_ref,
                 anch_ref, dc_ref, cb_ref, o_ref):
    sb = pl.program_id(1)
    t0 = sb * TA
    start = jnp.clip(t0 - HALO, 0, S - WA)

    q2 = q_ref[:, :] * SCALE
    kw2 = k_ref[pl.ds(start, WA), :]
    vw2 = v_ref[pl.ds(start, WA), :]

    qpos = (t0 + jax.lax.broadcasted_iota(jnp.int32, (TA, 1), 0)).astype(
        jnp.float32)
    jfw = (start + jax.lax.broadcasted_iota(jnp.int32, (1, WA), 1)
           ).astype(jnp.float32)
    lane = jax.lax.broadcasted_iota(jnp.int32, (1, 128), 1)
    elv2 = elw_ref[0]
    elq = elq_ref[:, :]

    out = None
    for hh in range(2):
        mask = ((lane >= 64 * hh) & (lane < 64 * (hh + 1))).astype(jnp.float32)
        off = off_ref[hh]
        ps = ps_ref[hh]

        sp = qpos - anch_ref[:, :] + off * OFFSET_SCALE
        sp = jnp.maximum(sp, 0.0)
        sp = jnp.minimum(sp, qpos)

        m_list = [
            jnp.maximum(1.0 - jnp.abs(jfw - sp[:, p:p + 1]), 0.0)
            for p in range(P)
        ]

        sd = jax.lax.dot_general(q2, kw2 * mask, (((1,), (1,)), ((), ())),
                                 preferred_element_type=jnp.float32)
        qk = _band_reduce(m_list, sd)
        sel = _band_reduce(m_list, elv2)
        rel = jnp.log1p(jnp.maximum(elq - sel, 0.0))

        logits = qk + ps + cb_ref[hh] - dc_ref[hh] * rel
        mx = jnp.max(logits, axis=1, keepdims=True)
        ex = jnp.exp(logits - mx)
        wgt = ex / jnp.sum(ex, axis=1, keepdims=True)

        a = wgt[:, 0:1] * m_list[0]
        for p in range(1, P):
            a = a + wgt[:, p:p + 1] * m_list[p]
        oh = jnp.dot(a, vw2 * mask, preferred_element_type=jnp.float32)
        out = oh if out is None else out + oh
    o_ref[:, :] = out


def _out_kernel(a_ref, w_ref, b_ref, o_ref):
    o_ref[:, :] = (
        jnp.dot(a_ref[:, :], w_ref[:, :], preferred_element_type=jnp.float32)
        + b_ref[:, :]
    )


def kernel(x, time_delta, Wq, bq, Wk, bk, Wv, bv, emb, Wdc, bdc, Wtc, btc,
           Woff, boff, Wps, bps, Wo, bo, point_bias, tdw, tdb):
    f32 = jnp.float32
    x2 = x[0]
    dt_raw = time_delta[0].reshape(S, 1)

    Wtc_x = Wtc[:D]
    Wtc_c = Wtc[D:D + DFD]
    Wtc_b = Wtc[D + DFD:]
    wdc_eff = (Wdc @ Wtc_c).reshape(1, 1, D)
    emb_eff = emb @ Wtc_b
    tc_bias = btc + bdc @ Wtc_c + emb_eff[0]
    emb_diff = (emb_eff[1:] - emb_eff[:-1]).reshape(8, 1, D)

    Wbig = jnp.concatenate([Wq, Wk, Wv, Wtc_x], axis=1)
    bias4 = jnp.stack([bq, bk, bv, tc_bias]).reshape(4, 1, D)

    qkvt = pl.pallas_call(
        _proj_kernel,
        grid=(4, SB),
        in_specs=[
            pl.BlockSpec((T, D), lambda nb, sb: (sb, 0)),
            pl.BlockSpec((D, D), lambda nb, sb: (0, nb)),
            pl.BlockSpec((1, 1, D), lambda nb, sb: (nb, 0, 0)),
        ],
        out_specs=pl.BlockSpec((T, D), lambda nb, sb: (sb, nb)),
        out_shape=jax.ShapeDtypeStruct((S, 4 * D), f32),
    )(x2, Wbig, bias4)

    bop = jnp.stack([boff, bps]).reshape(2, 1, H * P)
    off, pscore = pl.pallas_call(
        _tc_kernel,
        grid=(SB,),
        in_specs=[
            pl.BlockSpec((T, D), lambda sb: (sb, 3)),
            pl.BlockSpec((T, 1), lambda sb: (sb, 0)),
            pl.BlockSpec((1, 1, D), lambda sb: (0, 0, 0)),
            pl.BlockSpec((8, 1, D), lambda sb: (0, 0, 0)),
            pl.BlockSpec((D, H * P), lambda sb: (0, 0)),
            pl.BlockSpec((D, H * P), lambda sb: (0, 0)),
            pl.BlockSpec((2, 1, H * P), lambda sb: (0, 0, 0)),
        ],
        out_specs=[
            pl.BlockSpec((T, H * P), lambda sb: (sb, 0)),
            pl.BlockSpec((T, H * P), lambda sb: (sb, 0)),
        ],
        out_shape=[
            jax.ShapeDtypeStruct((S, H * P), f32),
            jax.ShapeDtypeStruct((S, H * P), f32),
        ],
    )(qkvt, dt_raw, wdc_eff, emb_diff, Woff, Wps, bop)

    off_t = off.reshape(S, H, P).transpose(1, 0, 2)
    ps_t = pscore.reshape(S, H, P).transpose(1, 0, 2)

    dtc = jnp.maximum(time_delta[0], 0.0)
    elapsed = jnp.cumsum(dtc)
    elq = elapsed.reshape(S, 1)
    starts = [min(max(0, TA * sb - HALO), S - WA) for sb in range(NA)]
    elw = jnp.stack(
        [elapsed[s0:s0 + WA] for s0 in starts]).reshape(NA, 1, WA)

    anchors = 2.0 ** jnp.linspace(0.0, math.log2(MAX_DIST + 1.0), P) - 1.0
    anchors = anchors.at[0].set(0.0).astype(f32)
    anch1 = anchors.reshape(1, P)
    dc = jax.nn.softplus(tdw).reshape(H, 1, P)
    cb = (tdb + point_bias).reshape(H, 1, P)

    attn = pl.pallas_call(
        _attn_kernel,
        grid=(H // 2, NA),
        in_specs=[
            pl.BlockSpec((TA, 128), lambda hp, sb: (sb, hp)),
            pl.BlockSpec((S, 128), lambda hp, sb: (0, 8 + hp)),
            pl.BlockSpec((S, 128), lambda hp, sb: (0, 16 + hp)),
            pl.BlockSpec((2, TA, P), lambda hp, sb: (hp, sb, 0)),
            pl.BlockSpec((2, TA, P), lambda hp, sb: (hp, sb, 0)),
            pl.BlockSpec((1, 1, WA), lambda hp, sb: (sb, 0, 0)),
            pl.BlockSpec((TA, 1), lambda hp, sb: (sb, 0)),
            pl.BlockSpec((1, P), lambda hp, sb: (0, 0)),
            pl.BlockSpec((2, 1, P), lambda hp, sb: (hp, 0, 0)),
            pl.BlockSpec((2, 1, P), lambda hp, sb: (hp, 0, 0)),
        ],
        out_specs=pl.BlockSpec((TA, 128), lambda hp, sb: (sb, hp)),
        out_shape=jax.ShapeDtypeStruct((S, D), f32),
    )(qkvt, qkvt, qkvt, off_t, ps_t, elw, elq, anch1, dc, cb)

    out = pl.pallas_call(
        _out_kernel,
        grid=(SB,),
        in_specs=[
            pl.BlockSpec((T, D), lambda sb: (sb, 0)),
            pl.BlockSpec((D, D), lambda sb: (0, 0)),
            pl.BlockSpec((1, D), lambda sb: (0, 0)),
        ],
        out_specs=pl.BlockSpec((T, D), lambda sb: (sb, 0)),
        out_shape=jax.ShapeDtypeStruct((S, D), f32),
    )(attn, Wo, bo.reshape(1, D))

    return out.reshape(1, S, D)
